# baseline (speedup 1.0000x reference)
import functools

import jax
import jax.numpy as jnp
from jax import lax
from jax.experimental import pallas as pl
from jax.experimental.pallas import tpu as pltpu
from jax.experimental.pallas import tpu_sc as plsc

NODE_DIM, EDGE_DIM, GLOBAL_DIM, HID, NT, STEPS = 12, 5, 11, 64, 4, 3
BN = 2000
BE = 3200


def _full(shape):
    return pl.BlockSpec(shape, lambda i: tuple(0 for _ in shape))


def _alpha_kernel(ea, A, c1, W2, b2):
    E = ea.shape[0]

    def body(ea_ref, A_ref, c1_ref, W2_ref, b2_ref, out_ref):
        t = jnp.dot(ea_ref[...], A_ref[...], preferred_element_type=jnp.float32)
        t = jnp.maximum(t + c1_ref[...], 0.0)
        z = jnp.dot(t, W2_ref[...], preferred_element_type=jnp.float32) + b2_ref[...]
        out_ref[...] = jax.nn.sigmoid(z)

    return pl.pallas_call(
        body,
        grid=(E // BE,),
        in_specs=[
            pl.BlockSpec((BE, EDGE_DIM), lambda i: (i, 0)),
            _full(A.shape), _full(c1.shape), _full(W2.shape), _full(b2.shape),
        ],
        out_specs=pl.BlockSpec((BE, 1), lambda i: (i, 0)),
        out_shape=jax.ShapeDtypeStruct((E, 1), jnp.float32),
    )(ea, A, c1, W2, b2)


def _node0_kernel(x, cnt, XT, c_in, WmT, b_m):
    n = x.shape[0]

    def body(x_ref, cnt_ref, XT_ref, cin_ref, WmT_ref, bm_ref,
             h_ref, g_ref, recip_ref):
        h = jnp.tanh(jnp.dot(x_ref[...], XT_ref[...],
                             preferred_element_type=jnp.float32) + cin_ref[...])
        g = jnp.maximum(jnp.dot(h, WmT_ref[...],
                                preferred_element_type=jnp.float32) + bm_ref[...], 0.0)
        h_ref[...] = h
        g_ref[0] = g[:, :32]
        g_ref[1] = g[:, 32:]
        recip_ref[...] = 1.0 / jnp.maximum(cnt_ref[...], 1.0)

    return pl.pallas_call(
        body,
        grid=(n // BN,),
        in_specs=[
            pl.BlockSpec((BN, NODE_DIM), lambda i: (i, 0)),
            pl.BlockSpec((BN, 1), lambda i: (i, 0)),
            _full(XT.shape), _full(c_in.shape), _full(WmT.shape), _full(b_m.shape),
        ],
        out_specs=[
            pl.BlockSpec((BN, HID), lambda i: (i, 0)),
            pl.BlockSpec((2, BN, 32), lambda i: (0, i, 0)),
            pl.BlockSpec((BN, 1), lambda i: (i, 0)),
        ],
        out_shape=[
            jax.ShapeDtypeStruct((n, HID), jnp.float32),
            jax.ShapeDtypeStruct((2, n, 32), jnp.float32),
            jax.ShapeDtypeStruct((n, 1), jnp.float32),
        ],
    )(x, cnt, XT, c_in, WmT, b_m)


def _gru_kernel(h, aggp, recip, WihT0, WihT1, b_ih, WhhT, b_hh, WmT, b_m,
                Wd1aT, Wd1bT, final):
    n = h.shape[0]

    def body(h_ref, agg_ref, recip_ref, WihT0_ref, WihT1_ref, bih_ref,
             WhhT_ref, bhh_ref, WmT_ref, bm_ref, Wa_ref, Wb_ref, *outs):
        r_ = recip_ref[...]
        a0 = agg_ref[0] * r_
        a1 = agg_ref[1] * r_
        gi = (jnp.dot(a0, WihT0_ref[...], preferred_element_type=jnp.float32)
              + jnp.dot(a1, WihT1_ref[...], preferred_element_type=jnp.float32)
              + bih_ref[...])
        h = h_ref[...]
        gh = jnp.dot(h, WhhT_ref[...], preferred_element_type=jnp.float32) + bhh_ref[...]
        r = jax.nn.sigmoid(gi[:, :HID] + gh[:, :HID])
        z = jax.nn.sigmoid(gi[:, HID:2 * HID] + gh[:, HID:2 * HID])
        nn_ = jnp.tanh(gi[:, 2 * HID:] + r * gh[:, 2 * HID:])
        h_new = (1.0 - z) * nn_ + z * h
        if final:
            pq_ref, = outs
            pq_ref[0] = jnp.dot(h_new, Wa_ref[...], preferred_element_type=jnp.float32)
            pq_ref[1] = jnp.dot(h_new, Wb_ref[...], preferred_element_type=jnp.float32)
        else:
            hn_ref, g_ref = outs
            hn_ref[...] = h_new
            g = jnp.maximum(jnp.dot(h_new, WmT_ref[...],
                                    preferred_element_type=jnp.float32) + bm_ref[...], 0.0)
            g_ref[0] = g[:, :32]
            g_ref[1] = g[:, 32:]

    if final:
        out_specs = [pl.BlockSpec((2, BN, HID), lambda i: (0, i, 0))]
        out_shape = [jax.ShapeDtypeStruct((2, n, HID), jnp.float32)]
    else:
        out_specs = [
            pl.BlockSpec((BN, HID), lambda i: (i, 0)),
            pl.BlockSpec((2, BN, 32), lambda i: (0, i, 0)),
        ]
        out_shape = [
            jax.ShapeDtypeStruct((n, HID), jnp.float32),
            jax.ShapeDtypeStruct((2, n, 32), jnp.float32),
        ]

    return pl.pallas_call(
        body,
        grid=(n // BN,),
        in_specs=[
            pl.BlockSpec((BN, HID), lambda i: (i, 0)),
            pl.BlockSpec((2, BN, 32), lambda i: (0, i, 0)),
            pl.BlockSpec((BN, 1), lambda i: (i, 0)),
            _full(WihT0.shape), _full(WihT1.shape), _full(b_ih.shape),
            _full(WhhT.shape), _full(b_hh.shape), _full(WmT.shape),
            _full(b_m.shape), _full(Wd1aT.shape), _full(Wd1bT.shape),
        ],
        out_specs=out_specs,
        out_shape=out_shape,
    )(h, aggp, recip, WihT0, WihT1, b_ih, WhhT, b_hh, WmT, b_m, Wd1aT, Wd1bT)


def _dec2_kernel(s, ea, WcT, b_d1, Wd2T, b_d2, Wd3T, b_d3):
    E = ea.shape[0]

    def body(s_ref, ea_ref, WcT_ref, b1_ref, W2_ref, b2_ref, W3_ref, b3_ref,
             out_ref):
        r_e = jnp.dot(ea_ref[...], WcT_ref[...], preferred_element_type=jnp.float32)
        d1 = jnp.maximum(s_ref[...] + r_e + b1_ref[...], 0.0)
        d2 = jnp.maximum(jnp.dot(d1, W2_ref[...],
                                 preferred_element_type=jnp.float32) + b2_ref[...], 0.0)
        out_ref[...] = jnp.dot(d2, W3_ref[...],
                               preferred_element_type=jnp.float32) + b3_ref[...]

    return pl.pallas_call(
        body,
        grid=(E // BE,),
        in_specs=[
            pl.BlockSpec((BE, HID), lambda i: (i, 0)),
            pl.BlockSpec((BE, EDGE_DIM), lambda i: (i, 0)),
            _full(WcT.shape), _full(b_d1.shape), _full(Wd2T.shape),
            _full(b_d2.shape), _full(Wd3T.shape), _full(b_d3.shape),
        ],
        out_specs=pl.BlockSpec((BE, NT), lambda i: (i, 0)),
        out_shape=jax.ShapeDtypeStruct((E, NT), jnp.float32),
    )(s, ea, WcT, b_d1, Wd2T, b_d2, Wd3T, b_d3)


def kernel(x, edge_index, edge_attr, u,
           W_in, b_in, W_e1, b_e1, W_e2, b_e2, W_m, b_m,
           W_ih, b_ih, W_hh, b_hh, W_d1, b_d1, W_d2, b_d2, W_d3, b_d3):
    n = x.shape[0]
    E = edge_attr.shape[0]
    src, dst = edge_index[0], edge_index[1]

    A = W_e1[:, :EDGE_DIM].T
    c1 = (b_e1 + (u @ W_e1[:, EDGE_DIM:].T)[0]).reshape(1, -1)
    W2 = W_e2.T
    b2 = b_e2.reshape(1, 1)
    XT = W_in[:, :NODE_DIM].T
    c_in = (b_in + (u @ W_in[:, NODE_DIM:].T)[0]).reshape(1, HID)
    WmT = W_m.T
    bm = b_m.reshape(1, HID)
    WihT0 = W_ih[:, :32].T
    WihT1 = W_ih[:, 32:].T
    bih = b_ih.reshape(1, 3 * HID)
    WhhT = W_hh.T
    bhh = b_hh.reshape(1, 3 * HID)
    Wd1aT = W_d1[:, :HID].T
    Wd1bT = W_d1[:, HID:2 * HID].T
    WcT = W_d1[:, 2 * HID:].T
    bd1 = b_d1.reshape(1, HID)
    Wd2T = W_d2.T
    bd2 = b_d2.reshape(1, -1)
    Wd3T = W_d3.T
    bd3 = b_d3.reshape(1, -1)

    alpha = _alpha_kernel(edge_attr, A, c1, W2, b2)

    cnt = jax.ops.segment_sum(jnp.ones((E,), jnp.float32), dst,
                              num_segments=n).reshape(n, 1)

    h, g, recip = _node0_kernel(x, cnt, XT, c_in, WmT, bm)

    for step in range(STEPS):
        g_cat = jnp.concatenate([g[0], g[1]], axis=-1)
        msg = alpha * g_cat[src]
        agg = jax.ops.segment_sum(msg, dst, num_segments=n)
        aggp = jnp.stack([agg[:, :32], agg[:, 32:]])
        final = step == STEPS - 1
        outs = _gru_kernel(h, aggp, recip, WihT0, WihT1, bih, WhhT, bhh,
                           WmT, bm, Wd1aT, Wd1bT, final)
        if final:
            pq, = outs
        else:
            h, g = outs

    s = pq[0][src] + pq[1][dst]
    return _dec2_kernel(s, edge_attr, WcT, bd1, Wd2T, bd2, Wd3T, bd3)


# speedup vs baseline: 4.7735x; 4.7735x over previous
import functools

import jax
import jax.numpy as jnp
from jax import lax
from jax.experimental import pallas as pl
from jax.experimental.pallas import tpu as pltpu
from jax.experimental.pallas import tpu_sc as plsc

NODE_DIM, EDGE_DIM, GLOBAL_DIM, HID, NT, STEPS = 12, 5, 11, 64, 4, 3
BN = 2000
BE = 3200
RE = 6272
NPAD = 50176


def _full(shape):
    return pl.BlockSpec(shape, lambda i: tuple(0 for _ in shape))


def _sc_mesh():
    return plsc.VectorSubcoreMesh(core_axis_name="c", subcore_axis_name="s")


def _alpha_kernel(ea, A, c1, W2, b2):
    E = ea.shape[0]

    def body(ea_ref, A_ref, c1_ref, W2_ref, b2_ref, out_ref):
        t = jnp.dot(ea_ref[...], A_ref[...], preferred_element_type=jnp.float32)
        t = jnp.maximum(t + c1_ref[...], 0.0)
        z = jnp.dot(t, W2_ref[...], preferred_element_type=jnp.float32) + b2_ref[...]
        out_ref[...] = jax.nn.sigmoid(z)

    return pl.pallas_call(
        body,
        grid=(E // BE,),
        in_specs=[
            pl.BlockSpec((BE, EDGE_DIM), lambda i: (i, 0)),
            _full(A.shape), _full(c1.shape), _full(W2.shape), _full(b2.shape),
        ],
        out_specs=pl.BlockSpec((BE, 1), lambda i: (i, 0)),
        out_shape=jax.ShapeDtypeStruct((E, 1), jnp.float32),
    )(ea, A, c1, W2, b2)


def _node0_kernel(x, cnt0, cnt1, XT, c_in, WmT, b_m):
    n = x.shape[0]

    def body(x_ref, c0_ref, c1r_ref, XT_ref, cin_ref, WmT_ref, bm_ref,
             h_ref, g_ref, recip_ref):
        h = jnp.tanh(jnp.dot(x_ref[...], XT_ref[...],
                             preferred_element_type=jnp.float32) + cin_ref[...])
        g = jnp.maximum(jnp.dot(h, WmT_ref[...],
                                preferred_element_type=jnp.float32) + bm_ref[...], 0.0)
        h_ref[...] = h
        g_ref[0] = g[:, :32]
        g_ref[1] = g[:, 32:]
        cnt = c0_ref[...] + c1r_ref[...]
        recip_ref[...] = 1.0 / jnp.maximum(cnt, 1.0)

    return pl.pallas_call(
        body,
        grid=(n // BN,),
        in_specs=[
            pl.BlockSpec((BN, NODE_DIM), lambda i: (i, 0)),
            pl.BlockSpec((BN, 1), lambda i: (i, 0)),
            pl.BlockSpec((BN, 1), lambda i: (i, 0)),
            _full(XT.shape), _full(c_in.shape), _full(WmT.shape), _full(b_m.shape),
        ],
        out_specs=[
            pl.BlockSpec((BN, HID), lambda i: (i, 0)),
            pl.BlockSpec((2, BN, 32), lambda i: (0, i, 0)),
            pl.BlockSpec((BN, 1), lambda i: (i, 0)),
        ],
        out_shape=[
            jax.ShapeDtypeStruct((n, HID), jnp.float32),
            jax.ShapeDtypeStruct((2, n, 32), jnp.float32),
            jax.ShapeDtypeStruct((n, 1), jnp.float32),
        ],
    )(x, cnt0, cnt1, XT, c_in, WmT, b_m)


def _gru_kernel(h, aggp, recip, WihT0, WihT1, b_ih, WhhT, b_hh, WmT, b_m,
                Wd1aT, Wd1bT, final):
    n = h.shape[0]

    def body(h_ref, agg_ref, recip_ref, WihT0_ref, WihT1_ref, bih_ref,
             WhhT_ref, bhh_ref, WmT_ref, bm_ref, Wa_ref, Wb_ref, *outs):
        r_ = recip_ref[...]
        a0 = agg_ref[0] * r_
        a1 = agg_ref[1] * r_
        gi = (jnp.dot(a0, WihT0_ref[...], preferred_element_type=jnp.float32)
              + jnp.dot(a1, WihT1_ref[...], preferred_element_type=jnp.float32)
              + bih_ref[...])
        h = h_ref[...]
        gh = jnp.dot(h, WhhT_ref[...], preferred_element_type=jnp.float32) + bhh_ref[...]
        r = jax.nn.sigmoid(gi[:, :HID] + gh[:, :HID])
        z = jax.nn.sigmoid(gi[:, HID:2 * HID] + gh[:, HID:2 * HID])
        nn_ = jnp.tanh(gi[:, 2 * HID:] + r * gh[:, 2 * HID:])
        h_new = (1.0 - z) * nn_ + z * h
        if final:
            pq_ref, = outs
            pq_ref[0] = jnp.dot(h_new, Wa_ref[...], preferred_element_type=jnp.float32)
            pq_ref[1] = jnp.dot(h_new, Wb_ref[...], preferred_element_type=jnp.float32)
        else:
            hn_ref, g_ref = outs
            hn_ref[...] = h_new
            g = jnp.maximum(jnp.dot(h_new, WmT_ref[...],
                                    preferred_element_type=jnp.float32) + bm_ref[...], 0.0)
            g_ref[0] = g[:, :32]
            g_ref[1] = g[:, 32:]

    if final:
        out_specs = [pl.BlockSpec((2, BN, HID), lambda i: (0, i, 0))]
        out_shape = [jax.ShapeDtypeStruct((2, n, HID), jnp.float32)]
    else:
        out_specs = [
            pl.BlockSpec((BN, HID), lambda i: (i, 0)),
            pl.BlockSpec((2, BN, 32), lambda i: (0, i, 0)),
        ]
        out_shape = [
            jax.ShapeDtypeStruct((n, HID), jnp.float32),
            jax.ShapeDtypeStruct((2, n, 32), jnp.float32),
        ]

    return pl.pallas_call(
        body,
        grid=(n // BN,),
        in_specs=[
            pl.BlockSpec((BN, HID), lambda i: (i, 0)),
            pl.BlockSpec((2, BN, 32), lambda i: (0, i, 0)),
            pl.BlockSpec((BN, 1), lambda i: (i, 0)),
            _full(WihT0.shape), _full(WihT1.shape), _full(b_ih.shape),
            _full(WhhT.shape), _full(b_hh.shape), _full(WmT.shape),
            _full(b_m.shape), _full(Wd1aT.shape), _full(Wd1bT.shape),
        ],
        out_specs=out_specs,
        out_shape=out_shape,
    )(h, aggp, recip, WihT0, WihT1, b_ih, WhhT, b_hh, WmT, b_m, Wd1aT, Wd1bT)


def _dec2_kernel(s, ea, WcT, b_d1, Wd2T, b_d2, Wd3T, b_d3):
    E = ea.shape[0]

    def body(s_ref, ea_ref, WcT_ref, b1_ref, W2_ref, b2_ref, W3_ref, b3_ref,
             out_ref):
        r_e = jnp.dot(ea_ref[...], WcT_ref[...], preferred_element_type=jnp.float32)
        d1 = jnp.maximum(s_ref[...] + r_e + b1_ref[...], 0.0)
        d2 = jnp.maximum(jnp.dot(d1, W2_ref[...],
                                 preferred_element_type=jnp.float32) + b2_ref[...], 0.0)
        out_ref[...] = jnp.dot(d2, W3_ref[...],
                               preferred_element_type=jnp.float32) + b3_ref[...]

    return pl.pallas_call(
        body,
        grid=(E // BE,),
        in_specs=[
            pl.BlockSpec((BE, HID), lambda i: (i, 0)),
            pl.BlockSpec((BE, EDGE_DIM), lambda i: (i, 0)),
            _full(WcT.shape), _full(b_d1.shape), _full(Wd2T.shape),
            _full(b_d2.shape), _full(Wd3T.shape), _full(b_d3.shape),
        ],
        out_specs=pl.BlockSpec((BE, NT), lambda i: (i, 0)),
        out_shape=jax.ShapeDtypeStruct((E, NT), jnp.float32),
    )(s, ea, WcT, b_d1, Wd2T, b_d2, Wd3T, b_d3)


def _zero16(buf, nvec):
    def zb(i, _):
        buf[pl.ds(i * 16, 16)] = jnp.zeros((16,), jnp.float32)
        return _
    lax.fori_loop(0, nvec, zb, None)


def _sc_cnt(dst2d, ones2d):
    rpw = RE // 32
    CH = 4
    nchunk = rpw // CH
    slab = NPAD // 16

    @functools.partial(
        pl.kernel,
        out_type=jax.ShapeDtypeStruct((2 * NPAD,), jnp.float32),
        mesh=_sc_mesh(),
        compiler_params=pltpu.CompilerParams(use_tc_tiling_on_sc=False),
        scratch_types=[
            pltpu.VMEM_SHARED((NPAD,), jnp.float32),
            pltpu.VMEM((CH, 128), jnp.int32),
            pltpu.VMEM((CH, 128), jnp.float32),
            pltpu.VMEM((slab,), jnp.float32),
        ],
    )
    def k(dst_hbm, ones_hbm, out_hbm, acc, idx_v, val_v, zbuf):
        c = lax.axis_index("c")
        s = lax.axis_index("s")
        _zero16(zbuf, slab // 16)
        pltpu.sync_copy(zbuf, acc.at[pl.ds(s * slab, slab)])
        plsc.subcore_barrier()
        row0 = (c * 16 + s) * rpw

        def chunk(kk, _):
            r0 = row0 + kk * CH
            pltpu.sync_copy(dst_hbm.at[pl.ds(r0, CH)], idx_v)
            pltpu.sync_copy(ones_hbm.at[pl.ds(r0, CH)], val_v)
            for j in range(CH):
                pltpu.sync_copy(val_v.at[j], acc.at[idx_v.at[j]], add=True)
            return _

        lax.fori_loop(0, nchunk, chunk, None)
        plsc.subcore_barrier()
        pltpu.sync_copy(acc.at[pl.ds(s * slab, slab)], zbuf)
        pltpu.sync_copy(zbuf, out_hbm.at[pl.ds(c * NPAD + s * slab, slab)])

    return k(dst2d, ones2d)


def _sc_spmm(pk2, alpha_p, g_flat, n):
    rpw = RE // 16
    CH = 2
    nchunk = rpw // CH
    slab = NPAD // 16
    ZR = 224

    @functools.partial(
        pl.kernel,
        out_type=jax.ShapeDtypeStruct((2 * NPAD, 32), jnp.float32),
        mesh=_sc_mesh(),
        compiler_params=pltpu.CompilerParams(use_tc_tiling_on_sc=False),
        scratch_types=[
            pltpu.VMEM_SHARED((NPAD, 32), jnp.float32),
            pltpu.VMEM((2, CH, 2, 128), jnp.int32),
            pltpu.VMEM((2, CH, 128), jnp.float32),
            pltpu.VMEM((2, CH * 128, 32), jnp.float32),
            pltpu.SemaphoreType.DMA,
            pltpu.SemaphoreType.DMA,
            pltpu.SemaphoreType.DMA,
            pltpu.SemaphoreType.DMA,
        ],
    )
    def k(pk_hbm, alpha_hbm, g_hbm, out_hbm, acc, pk_v, alpha_v, msg,
          semI, semA, semG, semS):
        c = lax.axis_index("c")
        s = lax.axis_index("s")

        def zm(i, _):
            msg[0, i, pl.ds(0, 16)] = jnp.zeros((16,), jnp.float32)
            msg[0, i, pl.ds(16, 16)] = jnp.zeros((16,), jnp.float32)
            return _
        lax.fori_loop(0, ZR, zm, None)

        def zcp(t, _):
            pltpu.sync_copy(msg.at[0, pl.ds(0, ZR)],
                            acc.at[pl.ds(s * slab + t * ZR, ZR)])
            return _
        lax.fori_loop(0, 14, zcp, None)
        plsc.subcore_barrier()

        coff = jnp.full((16,), c * n, jnp.int32)
        row0 = s * rpw

        def issue_idx(kk, b):
            pltpu.async_copy(pk_hbm.at[pl.ds(row0 + kk * CH, CH)],
                             pk_v.at[b], semI)
            pltpu.async_copy(alpha_hbm.at[pl.ds(row0 + kk * CH, CH)],
                             alpha_v.at[b], semA)

        def wait_idx(b):
            pltpu.make_async_copy(pk_hbm.at[pl.ds(0, CH)], pk_v.at[b],
                                  semI).wait()
            pltpu.make_async_copy(alpha_hbm.at[pl.ds(0, CH)], alpha_v.at[b],
                                  semA).wait()

        def offs(b):
            def ofr(i, _):
                sl = pl.ds(i * 16, 16)
                for r in range(CH):
                    pk_v[b, r, 0, sl] = pk_v[b, r, 0, sl] + coff
                return _
            lax.fori_loop(0, 8, ofr, None)

        def issue_gather(b):
            for r in range(CH):
                pltpu.async_copy(g_hbm.at[pk_v.at[b, r, 0]],
                                 msg.at[b, pl.ds(r * 128, 128)], semG)

        def wait_gather(b):
            for r in range(CH):
                pltpu.make_async_copy(g_hbm.at[pk_v.at[b, r, 0]],
                                      msg.at[b, pl.ds(r * 128, 128)],
                                      semG).wait()

        def scale(b):
            for r in range(CH):
                def srow(gi, _):
                    a16 = alpha_v[b, r, pl.ds(gi * 16, 16)]
                    for j in range(16):
                        a = a16[jnp.full((16,), j, jnp.int32)]
                        e = r * 128 + gi * 16 + j
                        msg[b, e, pl.ds(0, 16)] = msg[b, e, pl.ds(0, 16)] * a
                        msg[b, e, pl.ds(16, 16)] = msg[b, e, pl.ds(16, 16)] * a
                    return _
                lax.fori_loop(0, 8, srow, None)

        def scatter_sync(b):
            descs = [pltpu.async_copy(msg.at[b, pl.ds(r * 128, 128)],
                                      acc.at[pk_v.at[b, r, 1]], semS,
                                      add=True)
                     for r in range(CH)]
            for dd in descs:
                dd.wait()

        issue_idx(0, 0)
        issue_idx(1, 1)
        wait_idx(0)
        offs(0)
        issue_gather(0)

        @pl.loop(0, nchunk, step=2)
        def chunk_loop(k0):
            for b in range(2):
                kk = k0 + b
                b1 = 1 - b

                @pl.when(kk + 1 < nchunk)
                def _():
                    wait_idx(b1)
                    offs(b1)
                    issue_gather(b1)

                wait_gather(b)
                scale(b)
                scatter_sync(b)

                @pl.when(kk + 2 < nchunk)
                def _():
                    issue_idx(kk + 2, b)

        plsc.subcore_barrier()

        def ecp(t, _):
            pltpu.sync_copy(acc.at[pl.ds(s * slab + t * ZR, ZR)],
                            msg.at[0, pl.ds(0, ZR)])
            pltpu.sync_copy(msg.at[0, pl.ds(0, ZR)],
                            out_hbm.at[pl.ds(c * NPAD + s * slab + t * ZR, ZR)])
            return _
        lax.fori_loop(0, 14, ecp, None)

    return k(pk2, alpha_p, g_flat)


def _sc_dec(pk2, pq_flat):
    rpw = RE // 32
    CH = 2
    nchunk = rpw // CH

    @functools.partial(
        pl.kernel,
        out_type=jax.ShapeDtypeStruct((RE * 128, HID), jnp.float32),
        mesh=_sc_mesh(),
        compiler_params=pltpu.CompilerParams(use_tc_tiling_on_sc=False),
        scratch_types=[
            pltpu.VMEM((2, CH, 2, 128), jnp.int32),
            pltpu.VMEM((2, CH * 128, HID), jnp.float32),
            pltpu.VMEM((2, CH * 128, HID), jnp.float32),
            pltpu.SemaphoreType.DMA,
            pltpu.SemaphoreType.DMA,
        ],
    )
    def k(pk_hbm, pq_hbm, out_hbm, pk_v, bufp, bufq, semI, semG):
        c = lax.axis_index("c")
        s = lax.axis_index("s")
        row0 = (c * 16 + s) * rpw

        def issue_idx(kk, b):
            pltpu.async_copy(pk_hbm.at[pl.ds(row0 + kk * CH, CH)],
                             pk_v.at[b], semI)

        def wait_idx(b):
            pltpu.make_async_copy(pk_hbm.at[pl.ds(0, CH)], pk_v.at[b],
                                  semI).wait()

        def issue_gather(b):
            for r in range(CH):
                pltpu.async_copy(pq_hbm.at[pk_v.at[b, r, 0]],
                                 bufp.at[b, pl.ds(r * 128, 128)], semG)
                pltpu.async_copy(pq_hbm.at[pk_v.at[b, r, 1]],
                                 bufq.at[b, pl.ds(r * 128, 128)], semG)

        def wait_gather(b):
            for r in range(CH):
                pltpu.make_async_copy(pq_hbm.at[pk_v.at[b, r, 0]],
                                      bufp.at[b, pl.ds(r * 128, 128)],
                                      semG).wait()
                pltpu.make_async_copy(pq_hbm.at[pk_v.at[b, r, 1]],
                                      bufq.at[b, pl.ds(r * 128, 128)],
                                      semG).wait()

        def addbuf(b):
            def addb(i, _):
                for t in range(4):
                    sl = pl.ds(t * 16, 16)
                    bufp[b, i, sl] = bufp[b, i, sl] + bufq[b, i, sl]
                return _
            lax.fori_loop(0, CH * 128, addb, None)

        issue_idx(0, 0)
        issue_idx(1, 1)
        wait_idx(0)
        issue_gather(0)

        @pl.loop(0, nchunk, step=2)
        def chunk_loop(k0):
            for b in range(2):
                kk = k0 + b
                b1 = 1 - b

                @pl.when(kk + 1 < nchunk)
                def _():
                    wait_idx(b1)
                    issue_gather(b1)

                wait_gather(b)
                addbuf(b)
                pltpu.sync_copy(
                    bufp.at[b],
                    out_hbm.at[pl.ds((row0 + kk * CH) * 128, CH * 128)])

                @pl.when(kk + 2 < nchunk)
                def _():
                    issue_idx(kk + 2, b)

    return k(pk2, pq_flat)


def kernel(x, edge_index, edge_attr, u,
           W_in, b_in, W_e1, b_e1, W_e2, b_e2, W_m, b_m,
           W_ih, b_ih, W_hh, b_hh, W_d1, b_d1, W_d2, b_d2, W_d3, b_d3):
    n = x.shape[0]
    E = edge_attr.shape[0]
    src, dst = edge_index[0], edge_index[1]

    A = W_e1[:, :EDGE_DIM].T
    c1 = (b_e1 + (u @ W_e1[:, EDGE_DIM:].T)[0]).reshape(1, -1)
    W2 = W_e2.T
    b2 = b_e2.reshape(1, 1)
    XT = W_in[:, :NODE_DIM].T
    c_in = (b_in + (u @ W_in[:, NODE_DIM:].T)[0]).reshape(1, HID)
    WmT = W_m.T
    bm = b_m.reshape(1, HID)
    WihT0 = W_ih[:, :32].T
    WihT1 = W_ih[:, 32:].T
    bih = b_ih.reshape(1, 3 * HID)
    WhhT = W_hh.T
    bhh = b_hh.reshape(1, 3 * HID)
    Wd1aT = W_d1[:, :HID].T
    Wd1bT = W_d1[:, HID:2 * HID].T
    WcT = W_d1[:, 2 * HID:].T
    bd1 = b_d1.reshape(1, HID)
    Wd2T = W_d2.T
    bd2 = b_d2.reshape(1, -1)
    Wd3T = W_d3.T
    bd3 = b_d3.reshape(1, -1)

    pad = RE * 128 - E
    src_p = jnp.pad(src, (0, pad)).reshape(RE, 128)
    dst_p = jnp.pad(dst, (0, pad)).reshape(RE, 128)
    dstn_p = jnp.pad(dst + n, (0, pad)).reshape(RE, 128)
    ones_p = jnp.pad(jnp.ones((E,), jnp.float32), (0, pad)).reshape(RE, 128)

    alpha = _alpha_kernel(edge_attr, A, c1, W2, b2)
    alpha_p = jnp.pad(alpha[:, 0], (0, pad)).reshape(RE, 128)
    pk_sd = jnp.stack([src_p, dst_p], axis=1)
    pk2 = jnp.stack([src_p, dstn_p], axis=1)

    cnt2 = _sc_cnt(dst_p, ones_p).reshape(2, NPAD)
    cnt0 = cnt2[0, :n].reshape(n, 1)
    cnt1 = cnt2[1, :n].reshape(n, 1)
    h, g, recip = _node0_kernel(x, cnt0, cnt1, XT, c_in, WmT, bm)

    for step in range(STEPS):
        g_flat = g.reshape(2 * n, 32)
        aggp = _sc_spmm(pk_sd, alpha_p, g_flat, n).reshape(2, NPAD, 32)
        final = step == STEPS - 1
        outs = _gru_kernel(h, aggp, recip, WihT0, WihT1, bih, WhhT, bhh,
                           WmT, bm, Wd1aT, Wd1bT, final)
        if final:
            pq, = outs
        else:
            h, g = outs

    pq_flat = pq.reshape(2 * n, HID)
    s = _sc_dec(pk2, pq_flat)
    return _dec2_kernel(s, edge_attr, WcT, bd1, Wd2T, bd2, Wd3T, bd3)


# speedup vs baseline: 5.6236x; 1.1781x over previous
import functools

import jax
import jax.numpy as jnp
from jax import lax
from jax.experimental import pallas as pl
from jax.experimental.pallas import tpu as pltpu
from jax.experimental.pallas import tpu_sc as plsc

NODE_DIM, EDGE_DIM, GLOBAL_DIM, HID, NT, STEPS = 12, 5, 11, 64, 4, 3
BN = 2000
BE = 3200
RE = 6272
NPAD = 50176


def _full(shape):
    return pl.BlockSpec(shape, lambda i: tuple(0 for _ in shape))


def _sc_mesh():
    return plsc.VectorSubcoreMesh(core_axis_name="c", subcore_axis_name="s")


def _alpha_kernel(eaT, A2, c1c, w2c, b2):
    E = eaT.shape[1]

    def body(eaT_ref, A2_ref, c1_ref, w2_ref, b2_ref, out_ref):
        t = jnp.dot(A2_ref[...], eaT_ref[...], preferred_element_type=jnp.float32)
        t = jnp.maximum(t + c1_ref[...], 0.0)
        z = jnp.sum(t * w2_ref[...], axis=0, keepdims=True) + b2_ref[...]
        out_ref[...] = jax.nn.sigmoid(z)

    return pl.pallas_call(
        body,
        grid=(E // BE,),
        in_specs=[
            pl.BlockSpec((EDGE_DIM, BE), lambda i: (0, i)),
            _full(A2.shape), _full(c1c.shape), _full(w2c.shape), _full(b2.shape),
        ],
        out_specs=pl.BlockSpec((1, BE), lambda i: (0, i)),
        out_shape=jax.ShapeDtypeStruct((1, E), jnp.float32),
    )(eaT, A2, c1c, w2c, b2)


def _node0_kernel(x, cnt0, cnt1, XT, c_in, WmT, b_m):
    n = x.shape[0]

    def body(x_ref, c0_ref, c1r_ref, XT_ref, cin_ref, WmT_ref, bm_ref,
             h_ref, g_ref, recip_ref):
        h = jnp.tanh(jnp.dot(x_ref[...], XT_ref[...],
                             preferred_element_type=jnp.float32) + cin_ref[...])
        g = jnp.maximum(jnp.dot(h, WmT_ref[...],
                                preferred_element_type=jnp.float32) + bm_ref[...], 0.0)
        h_ref[...] = h
        g_ref[0] = g[:, :32]
        g_ref[1] = g[:, 32:]
        cnt = c0_ref[...] + c1r_ref[...]
        recip_ref[...] = 1.0 / jnp.maximum(cnt, 1.0)

    return pl.pallas_call(
        body,
        grid=(n // BN,),
        in_specs=[
            pl.BlockSpec((BN, NODE_DIM), lambda i: (i, 0)),
            pl.BlockSpec((BN, 1), lambda i: (i, 0)),
            pl.BlockSpec((BN, 1), lambda i: (i, 0)),
            _full(XT.shape), _full(c_in.shape), _full(WmT.shape), _full(b_m.shape),
        ],
        out_specs=[
            pl.BlockSpec((BN, HID), lambda i: (i, 0)),
            pl.BlockSpec((2, BN, 32), lambda i: (0, i, 0)),
            pl.BlockSpec((BN, 1), lambda i: (i, 0)),
        ],
        out_shape=[
            jax.ShapeDtypeStruct((n, HID), jnp.float32),
            jax.ShapeDtypeStruct((2, n, 32), jnp.float32),
            jax.ShapeDtypeStruct((n, 1), jnp.float32),
        ],
    )(x, cnt0, cnt1, XT, c_in, WmT, b_m)


def _gru_kernel(h, aggp, recip, WihT0, WihT1, b_ih, WhhT, b_hh, WmT, b_m,
                Wd1aT, Wd1bT, final):
    n = h.shape[0]

    def body(h_ref, agg_ref, recip_ref, WihT0_ref, WihT1_ref, bih_ref,
             WhhT_ref, bhh_ref, WmT_ref, bm_ref, Wa_ref, Wb_ref, *outs):
        r_ = recip_ref[...]
        a0 = agg_ref[0] * r_
        a1 = agg_ref[1] * r_
        gi = (jnp.dot(a0, WihT0_ref[...], preferred_element_type=jnp.float32)
              + jnp.dot(a1, WihT1_ref[...], preferred_element_type=jnp.float32)
              + bih_ref[...])
        h = h_ref[...]
        gh = jnp.dot(h, WhhT_ref[...], preferred_element_type=jnp.float32) + bhh_ref[...]
        r = jax.nn.sigmoid(gi[:, :HID] + gh[:, :HID])
        z = jax.nn.sigmoid(gi[:, HID:2 * HID] + gh[:, HID:2 * HID])
        nn_ = jnp.tanh(gi[:, 2 * HID:] + r * gh[:, 2 * HID:])
        h_new = (1.0 - z) * nn_ + z * h
        if final:
            pq_ref, = outs
            pq_ref[0] = jnp.dot(h_new, Wa_ref[...], preferred_element_type=jnp.float32)
            pq_ref[1] = jnp.dot(h_new, Wb_ref[...], preferred_element_type=jnp.float32)
        else:
            hn_ref, g_ref = outs
            hn_ref[...] = h_new
            g = jnp.maximum(jnp.dot(h_new, WmT_ref[...],
                                    preferred_element_type=jnp.float32) + bm_ref[...], 0.0)
            g_ref[0] = g[:, :32]
            g_ref[1] = g[:, 32:]

    if final:
        out_specs = [pl.BlockSpec((2, BN, HID), lambda i: (0, i, 0))]
        out_shape = [jax.ShapeDtypeStruct((2, n, HID), jnp.float32)]
    else:
        out_specs = [
            pl.BlockSpec((BN, HID), lambda i: (i, 0)),
            pl.BlockSpec((2, BN, 32), lambda i: (0, i, 0)),
        ]
        out_shape = [
            jax.ShapeDtypeStruct((n, HID), jnp.float32),
            jax.ShapeDtypeStruct((2, n, 32), jnp.float32),
        ]

    return pl.pallas_call(
        body,
        grid=(n // BN,),
        in_specs=[
            pl.BlockSpec((BN, HID), lambda i: (i, 0)),
            pl.BlockSpec((2, BN, 32), lambda i: (0, i, 0)),
            pl.BlockSpec((BN, 1), lambda i: (i, 0)),
            _full(WihT0.shape), _full(WihT1.shape), _full(b_ih.shape),
            _full(WhhT.shape), _full(b_hh.shape), _full(WmT.shape),
            _full(b_m.shape), _full(Wd1aT.shape), _full(Wd1bT.shape),
        ],
        out_specs=out_specs,
        out_shape=out_shape,
    )(h, aggp, recip, WihT0, WihT1, b_ih, WhhT, b_hh, WmT, b_m, Wd1aT, Wd1bT)


def _dec2_kernel(s, eaT, WcT, b_d1, Wd2T, b_d2, Wd3T, b_d3):
    E = eaT.shape[1]

    def body(s_ref, eaT_ref, WcT_ref, b1_ref, W2_ref, b2_ref, W3_ref, b3_ref,
             out_ref):
        r_e = lax.dot_general(eaT_ref[...], WcT_ref[...],
                              (((0,), (0,)), ((), ())),
                              preferred_element_type=jnp.float32)
        d1 = jnp.maximum(s_ref[...] + r_e + b1_ref[...], 0.0)
        d2 = jnp.maximum(jnp.dot(d1, W2_ref[...],
                                 preferred_element_type=jnp.float32) + b2_ref[...], 0.0)
        out_ref[...] = jnp.dot(d2, W3_ref[...],
                               preferred_element_type=jnp.float32) + b3_ref[...]

    return pl.pallas_call(
        body,
        grid=(E // BE,),
        in_specs=[
            pl.BlockSpec((BE, HID), lambda i: (i, 0)),
            pl.BlockSpec((EDGE_DIM, BE), lambda i: (0, i)),
            _full(WcT.shape), _full(b_d1.shape), _full(Wd2T.shape),
            _full(b_d2.shape), _full(Wd3T.shape), _full(b_d3.shape),
        ],
        out_specs=pl.BlockSpec((BE, NT), lambda i: (i, 0)),
        out_shape=jax.ShapeDtypeStruct((E, NT), jnp.float32),
    )(s, eaT, WcT, b_d1, Wd2T, b_d2, Wd3T, b_d3)


def _zero16(buf, nvec):
    def zb(i, _):
        buf[pl.ds(i * 16, 16)] = jnp.zeros((16,), jnp.float32)
        return _
    lax.fori_loop(0, nvec, zb, None)


def _sc_cnt(dst2d, ones2d):
    rpw = RE // 32
    CH = 4
    nchunk = rpw // CH
    slab = NPAD // 16

    @functools.partial(
        pl.kernel,
        out_type=jax.ShapeDtypeStruct((2 * NPAD,), jnp.float32),
        mesh=_sc_mesh(),
        compiler_params=pltpu.CompilerParams(use_tc_tiling_on_sc=False),
        scratch_types=[
            pltpu.VMEM_SHARED((NPAD,), jnp.float32),
            pltpu.VMEM((CH, 128), jnp.int32),
            pltpu.VMEM((CH, 128), jnp.float32),
            pltpu.VMEM((slab,), jnp.float32),
        ],
    )
    def k(dst_hbm, ones_hbm, out_hbm, acc, idx_v, val_v, zbuf):
        c = lax.axis_index("c")
        s = lax.axis_index("s")
        _zero16(zbuf, slab // 16)
        pltpu.sync_copy(zbuf, acc.at[pl.ds(s * slab, slab)])
        plsc.subcore_barrier()
        row0 = (c * 16 + s) * rpw

        def chunk(kk, _):
            r0 = row0 + kk * CH
            pltpu.sync_copy(dst_hbm.at[pl.ds(r0, CH)], idx_v)
            pltpu.sync_copy(ones_hbm.at[pl.ds(r0, CH)], val_v)
            for j in range(CH):
                pltpu.sync_copy(val_v.at[j], acc.at[idx_v.at[j]], add=True)
            return _

        lax.fori_loop(0, nchunk, chunk, None)
        plsc.subcore_barrier()
        pltpu.sync_copy(acc.at[pl.ds(s * slab, slab)], zbuf)
        pltpu.sync_copy(zbuf, out_hbm.at[pl.ds(c * NPAD + s * slab, slab)])

    return k(dst2d, ones2d)


def _sc_spmm(pk2, alpha_p, g_flat, n):
    rpw = RE // 16
    CH = 2
    nchunk = rpw // CH
    slab = NPAD // 16
    ZR = 224

    @functools.partial(
        pl.kernel,
        out_type=jax.ShapeDtypeStruct((2 * NPAD, 32), jnp.float32),
        mesh=_sc_mesh(),
        compiler_params=pltpu.CompilerParams(use_tc_tiling_on_sc=False),
        scratch_types=[
            pltpu.VMEM_SHARED((NPAD, 32), jnp.float32),
            pltpu.VMEM((2, CH, 2, 128), jnp.int32),
            pltpu.VMEM((2, CH, 128), jnp.float32),
            pltpu.VMEM((2, CH * 128, 32), jnp.float32),
            pltpu.SemaphoreType.DMA,
            pltpu.SemaphoreType.DMA,
            pltpu.SemaphoreType.DMA,
            pltpu.SemaphoreType.DMA,
        ],
    )
    def k(pk_hbm, alpha_hbm, g_hbm, out_hbm, acc, pk_v, alpha_v, msg,
          semI, semA, semG, semS):
        c = lax.axis_index("c")
        s = lax.axis_index("s")

        def zm(i, _):
            msg[0, i, pl.ds(0, 16)] = jnp.zeros((16,), jnp.float32)
            msg[0, i, pl.ds(16, 16)] = jnp.zeros((16,), jnp.float32)
            return _
        lax.fori_loop(0, ZR, zm, None)

        def zcp(t, _):
            pltpu.sync_copy(msg.at[0, pl.ds(0, ZR)],
                            acc.at[pl.ds(s * slab + t * ZR, ZR)])
            return _
        lax.fori_loop(0, 14, zcp, None)
        plsc.subcore_barrier()

        coff = jnp.full((16,), c * n, jnp.int32)
        row0 = s * rpw

        def issue_idx(kk, b):
            pltpu.async_copy(pk_hbm.at[pl.ds(row0 + kk * CH, CH)],
                             pk_v.at[b], semI)
            pltpu.async_copy(alpha_hbm.at[pl.ds(row0 + kk * CH, CH)],
                             alpha_v.at[b], semA)

        def wait_idx(b):
            pltpu.make_async_copy(pk_hbm.at[pl.ds(0, CH)], pk_v.at[b],
                                  semI).wait()
            pltpu.make_async_copy(alpha_hbm.at[pl.ds(0, CH)], alpha_v.at[b],
                                  semA).wait()

        def offs(b):
            def ofr(i, _):
                sl = pl.ds(i * 16, 16)
                for r in range(CH):
                    pk_v[b, r, 0, sl] = pk_v[b, r, 0, sl] + coff
                return _
            lax.fori_loop(0, 8, ofr, None)

        def issue_gather(b):
            for r in range(CH):
                pltpu.async_copy(g_hbm.at[pk_v.at[b, r, 0]],
                                 msg.at[b, pl.ds(r * 128, 128)], semG)

        def wait_gather(b):
            for r in range(CH):
                pltpu.make_async_copy(g_hbm.at[pk_v.at[b, r, 0]],
                                      msg.at[b, pl.ds(r * 128, 128)],
                                      semG).wait()

        def scale(b):
            for r in range(CH):
                def srow(gi, _):
                    a16 = alpha_v[b, r, pl.ds(gi * 16, 16)]
                    for j in range(16):
                        a = a16[jnp.full((16,), j, jnp.int32)]
                        e = r * 128 + gi * 16 + j
                        msg[b, e, pl.ds(0, 16)] = msg[b, e, pl.ds(0, 16)] * a
                        msg[b, e, pl.ds(16, 16)] = msg[b, e, pl.ds(16, 16)] * a
                    return _
                lax.fori_loop(0, 8, srow, None)

        def scatter_sync(b):
            descs = [pltpu.async_copy(msg.at[b, pl.ds(r * 128, 128)],
                                      acc.at[pk_v.at[b, r, 1]], semS,
                                      add=True)
                     for r in range(CH)]
            for dd in descs:
                dd.wait()

        issue_idx(0, 0)
        issue_idx(1, 1)
        wait_idx(0)
        offs(0)
        issue_gather(0)

        @pl.loop(0, nchunk, step=2)
        def chunk_loop(k0):
            for b in range(2):
                kk = k0 + b
                b1 = 1 - b

                @pl.when(kk + 1 < nchunk)
                def _():
                    wait_idx(b1)
                    offs(b1)
                    issue_gather(b1)

                wait_gather(b)
                scale(b)
                scatter_sync(b)

                @pl.when(kk + 2 < nchunk)
                def _():
                    issue_idx(kk + 2, b)

        plsc.subcore_barrier()

        def ecp(t, _):
            pltpu.sync_copy(acc.at[pl.ds(s * slab + t * ZR, ZR)],
                            msg.at[0, pl.ds(0, ZR)])
            pltpu.sync_copy(msg.at[0, pl.ds(0, ZR)],
                            out_hbm.at[pl.ds(c * NPAD + s * slab + t * ZR, ZR)])
            return _
        lax.fori_loop(0, 14, ecp, None)

    return k(pk2, alpha_p, g_flat)


def _sc_dec(pk2, pq_flat):
    rpw = RE // 32
    CH = 2
    nchunk = rpw // CH

    @functools.partial(
        pl.kernel,
        out_type=jax.ShapeDtypeStruct((RE * 128, HID), jnp.float32),
        mesh=_sc_mesh(),
        compiler_params=pltpu.CompilerParams(use_tc_tiling_on_sc=False),
        scratch_types=[
            pltpu.VMEM((2, CH, 2, 128), jnp.int32),
            pltpu.VMEM((2, CH * 128, HID), jnp.float32),
            pltpu.VMEM((2, CH * 128, HID), jnp.float32),
            pltpu.SemaphoreType.DMA,
            pltpu.SemaphoreType.DMA,
        ],
    )
    def k(pk_hbm, pq_hbm, out_hbm, pk_v, bufp, bufq, semI, semG):
        c = lax.axis_index("c")
        s = lax.axis_index("s")
        row0 = (c * 16 + s) * rpw

        def issue_idx(kk, b):
            pltpu.async_copy(pk_hbm.at[pl.ds(row0 + kk * CH, CH)],
                             pk_v.at[b], semI)

        def wait_idx(b):
            pltpu.make_async_copy(pk_hbm.at[pl.ds(0, CH)], pk_v.at[b],
                                  semI).wait()

        def issue_gather(b):
            for r in range(CH):
                pltpu.async_copy(pq_hbm.at[pk_v.at[b, r, 0]],
                                 bufp.at[b, pl.ds(r * 128, 128)], semG)
                pltpu.async_copy(pq_hbm.at[pk_v.at[b, r, 1]],
                                 bufq.at[b, pl.ds(r * 128, 128)], semG)

        def wait_gather(b):
            for r in range(CH):
                pltpu.make_async_copy(pq_hbm.at[pk_v.at[b, r, 0]],
                                      bufp.at[b, pl.ds(r * 128, 128)],
                                      semG).wait()
                pltpu.make_async_copy(pq_hbm.at[pk_v.at[b, r, 1]],
                                      bufq.at[b, pl.ds(r * 128, 128)],
                                      semG).wait()

        def addbuf(b):
            def addb(i, _):
                for t in range(4):
                    sl = pl.ds(t * 16, 16)
                    bufp[b, i, sl] = bufp[b, i, sl] + bufq[b, i, sl]
                return _
            lax.fori_loop(0, CH * 128, addb, None)

        issue_idx(0, 0)
        issue_idx(1, 1)
        wait_idx(0)
        issue_gather(0)

        @pl.loop(0, nchunk, step=2)
        def chunk_loop(k0):
            for b in range(2):
                kk = k0 + b
                b1 = 1 - b

                @pl.when(kk + 1 < nchunk)
                def _():
                    wait_idx(b1)
                    issue_gather(b1)

                wait_gather(b)
                addbuf(b)
                pltpu.sync_copy(
                    bufp.at[b],
                    out_hbm.at[pl.ds((row0 + kk * CH) * 128, CH * 128)])

                @pl.when(kk + 2 < nchunk)
                def _():
                    issue_idx(kk + 2, b)

    return k(pk2, pq_flat)


def kernel(x, edge_index, edge_attr, u,
           W_in, b_in, W_e1, b_e1, W_e2, b_e2, W_m, b_m,
           W_ih, b_ih, W_hh, b_hh, W_d1, b_d1, W_d2, b_d2, W_d3, b_d3):
    n = x.shape[0]
    E = edge_attr.shape[0]
    src, dst = edge_index[0], edge_index[1]

    A2 = W_e1[:, :EDGE_DIM]
    c1c = (b_e1 + (u @ W_e1[:, EDGE_DIM:].T)[0]).reshape(-1, 1)
    w2c = W_e2.T
    b2 = b_e2.reshape(1, 1)
    XT = W_in[:, :NODE_DIM].T
    c_in = (b_in + (u @ W_in[:, NODE_DIM:].T)[0]).reshape(1, HID)
    WmT = W_m.T
    bm = b_m.reshape(1, HID)
    WihT0 = W_ih[:, :32].T
    WihT1 = W_ih[:, 32:].T
    bih = b_ih.reshape(1, 3 * HID)
    WhhT = W_hh.T
    bhh = b_hh.reshape(1, 3 * HID)
    Wd1aT = W_d1[:, :HID].T
    Wd1bT = W_d1[:, HID:2 * HID].T
    WcT = W_d1[:, 2 * HID:].T
    bd1 = b_d1.reshape(1, HID)
    Wd2T = W_d2.T
    bd2 = b_d2.reshape(1, -1)
    Wd3T = W_d3.T
    bd3 = b_d3.reshape(1, -1)

    pad = RE * 128 - E
    src_p = jnp.pad(src, (0, pad)).reshape(RE, 128)
    dst_p = jnp.pad(dst, (0, pad)).reshape(RE, 128)
    dstn_p = jnp.pad(dst + n, (0, pad)).reshape(RE, 128)
    ones_p = jnp.pad(jnp.ones((E,), jnp.float32), (0, pad)).reshape(RE, 128)

    eaT = edge_attr.T
    alpha = _alpha_kernel(eaT, A2, c1c, w2c, b2)
    alpha_p = jnp.pad(alpha[0], (0, pad)).reshape(RE, 128)
    pk_sd = jnp.stack([src_p, dst_p], axis=1)
    pk2 = jnp.stack([src_p, dstn_p], axis=1)

    cnt2 = _sc_cnt(dst_p, ones_p).reshape(2, NPAD)
    cnt0 = cnt2[0, :n].reshape(n, 1)
    cnt1 = cnt2[1, :n].reshape(n, 1)
    h, g, recip = _node0_kernel(x, cnt0, cnt1, XT, c_in, WmT, bm)

    for step in range(STEPS):
        g_flat = g.reshape(2 * n, 32)
        aggp = _sc_spmm(pk_sd, alpha_p, g_flat, n).reshape(2, NPAD, 32)
        final = step == STEPS - 1
        outs = _gru_kernel(h, aggp, recip, WihT0, WihT1, bih, WhhT, bhh,
                           WmT, bm, Wd1aT, Wd1bT, final)
        if final:
            pq, = outs
        else:
            h, g = outs

    pq_flat = pq.reshape(2 * n, HID)
    s = _sc_dec(pk2, pq_flat)
    return _dec2_kernel(s, eaT, WcT, bd1, Wd2T, bd2, Wd3T, bd3)


# speedup vs baseline: 6.0390x; 1.0739x over previous
import functools

import jax
import jax.numpy as jnp
from jax import lax
from jax.experimental import pallas as pl
from jax.experimental.pallas import tpu as pltpu
from jax.experimental.pallas import tpu_sc as plsc

NODE_DIM, EDGE_DIM, GLOBAL_DIM, HID, NT, STEPS = 12, 5, 11, 64, 4, 3
BN = 2000
BE = 3200
RE = 6272
NPAD = 50176


def _full(shape):
    return pl.BlockSpec(shape, lambda i: tuple(0 for _ in shape))


def _sc_mesh():
    return plsc.VectorSubcoreMesh(core_axis_name="c", subcore_axis_name="s")


def _alpha_kernel(eaT, A2, c1c, w2c, b2):
    E = eaT.shape[1]

    def body(eaT_ref, A2_ref, c1_ref, w2_ref, b2_ref, out_ref):
        t = jnp.dot(A2_ref[...], eaT_ref[...], preferred_element_type=jnp.float32)
        t = jnp.maximum(t + c1_ref[...], 0.0)
        z = jnp.sum(t * w2_ref[...], axis=0, keepdims=True) + b2_ref[...]
        out_ref[...] = jax.nn.sigmoid(z)

    return pl.pallas_call(
        body,
        grid=(E // BE,),
        in_specs=[
            pl.BlockSpec((EDGE_DIM, BE), lambda i: (0, i)),
            _full(A2.shape), _full(c1c.shape), _full(w2c.shape), _full(b2.shape),
        ],
        out_specs=pl.BlockSpec((1, BE), lambda i: (0, i)),
        out_shape=jax.ShapeDtypeStruct((1, E), jnp.float32),
    )(eaT, A2, c1c, w2c, b2)


def _node0_kernel(x, cnt0, cnt1, XT, c_in, WmT, b_m):
    n = x.shape[0]

    def body(x_ref, c0_ref, c1r_ref, XT_ref, cin_ref, WmT_ref, bm_ref,
             h_ref, g_ref, recip_ref):
        h = jnp.tanh(jnp.dot(x_ref[...], XT_ref[...],
                             preferred_element_type=jnp.float32) + cin_ref[...])
        g = jnp.maximum(jnp.dot(h, WmT_ref[...],
                                preferred_element_type=jnp.float32) + bm_ref[...], 0.0)
        h_ref[...] = h
        g_ref[0] = g[:, :32]
        g_ref[1] = g[:, 32:]
        cnt = c0_ref[...] + c1r_ref[...]
        recip_ref[...] = 1.0 / jnp.maximum(cnt, 1.0)

    return pl.pallas_call(
        body,
        grid=(n // BN,),
        in_specs=[
            pl.BlockSpec((BN, NODE_DIM), lambda i: (i, 0)),
            pl.BlockSpec((BN, 1), lambda i: (i, 0)),
            pl.BlockSpec((BN, 1), lambda i: (i, 0)),
            _full(XT.shape), _full(c_in.shape), _full(WmT.shape), _full(b_m.shape),
        ],
        out_specs=[
            pl.BlockSpec((BN, HID), lambda i: (i, 0)),
            pl.BlockSpec((2, BN, 32), lambda i: (0, i, 0)),
            pl.BlockSpec((BN, 1), lambda i: (i, 0)),
        ],
        out_shape=[
            jax.ShapeDtypeStruct((n, HID), jnp.float32),
            jax.ShapeDtypeStruct((2, n, 32), jnp.float32),
            jax.ShapeDtypeStruct((n, 1), jnp.float32),
        ],
    )(x, cnt0, cnt1, XT, c_in, WmT, b_m)


def _gru_kernel(h, aggp, recip, WihT0, WihT1, b_ih, WhhT, b_hh, WmT, b_m,
                Wd1aT, Wd1bT, final):
    n = h.shape[0]

    def body(h_ref, agg_ref, recip_ref, WihT0_ref, WihT1_ref, bih_ref,
             WhhT_ref, bhh_ref, WmT_ref, bm_ref, Wa_ref, Wb_ref, *outs):
        r_ = recip_ref[...]
        a0 = agg_ref[0] * r_
        a1 = agg_ref[1] * r_
        gi = (jnp.dot(a0, WihT0_ref[...], preferred_element_type=jnp.float32)
              + jnp.dot(a1, WihT1_ref[...], preferred_element_type=jnp.float32)
              + bih_ref[...])
        h = h_ref[...]
        gh = jnp.dot(h, WhhT_ref[...], preferred_element_type=jnp.float32) + bhh_ref[...]
        r = jax.nn.sigmoid(gi[:, :HID] + gh[:, :HID])
        z = jax.nn.sigmoid(gi[:, HID:2 * HID] + gh[:, HID:2 * HID])
        nn_ = jnp.tanh(gi[:, 2 * HID:] + r * gh[:, 2 * HID:])
        h_new = (1.0 - z) * nn_ + z * h
        if final:
            pq_ref, = outs
            pq_ref[0] = jnp.dot(h_new, Wa_ref[...], preferred_element_type=jnp.float32)
            pq_ref[1] = jnp.dot(h_new, Wb_ref[...], preferred_element_type=jnp.float32)
        else:
            hn_ref, g_ref = outs
            hn_ref[...] = h_new
            g = jnp.maximum(jnp.dot(h_new, WmT_ref[...],
                                    preferred_element_type=jnp.float32) + bm_ref[...], 0.0)
            g_ref[0] = g[:, :32]
            g_ref[1] = g[:, 32:]

    if final:
        out_specs = [pl.BlockSpec((2, BN, HID), lambda i: (0, i, 0))]
        out_shape = [jax.ShapeDtypeStruct((2, n, HID), jnp.float32)]
    else:
        out_specs = [
            pl.BlockSpec((BN, HID), lambda i: (i, 0)),
            pl.BlockSpec((2, BN, 32), lambda i: (0, i, 0)),
        ]
        out_shape = [
            jax.ShapeDtypeStruct((n, HID), jnp.float32),
            jax.ShapeDtypeStruct((2, n, 32), jnp.float32),
        ]

    return pl.pallas_call(
        body,
        grid=(n // BN,),
        in_specs=[
            pl.BlockSpec((BN, HID), lambda i: (i, 0)),
            pl.BlockSpec((2, BN, 32), lambda i: (0, i, 0)),
            pl.BlockSpec((BN, 1), lambda i: (i, 0)),
            _full(WihT0.shape), _full(WihT1.shape), _full(b_ih.shape),
            _full(WhhT.shape), _full(b_hh.shape), _full(WmT.shape),
            _full(b_m.shape), _full(Wd1aT.shape), _full(Wd1bT.shape),
        ],
        out_specs=out_specs,
        out_shape=out_shape,
    )(h, aggp, recip, WihT0, WihT1, b_ih, WhhT, b_hh, WmT, b_m, Wd1aT, Wd1bT)


def _dec2_kernel(s_pair, eaTd, Wcp, b1p, W2p, b2p, W3p, b3p, E):
    BE2 = 3200

    def body(s_ref, ead_ref, Wcp_ref, b1_ref, W2_ref, b2_ref, W3_ref, b3_ref,
             out_ref):
        rp = lax.dot_general(ead_ref[...], Wcp_ref[...],
                             (((0,), (0,)), ((), ())),
                             preferred_element_type=jnp.float32)
        d1 = jnp.maximum(s_ref[...] + rp + b1_ref[...], 0.0)
        d2 = jnp.maximum(jnp.dot(d1, W2_ref[...],
                                 preferred_element_type=jnp.float32) + b2_ref[...], 0.0)
        out_ref[...] = jnp.dot(d2, W3_ref[...],
                               preferred_element_type=jnp.float32) + b3_ref[...]

    return pl.pallas_call(
        body,
        grid=(E // (2 * BE2),),
        in_specs=[
            pl.BlockSpec((BE2, 128), lambda i: (i, 0)),
            pl.BlockSpec((2 * EDGE_DIM, BE2), lambda i: (0, i)),
            _full(Wcp.shape), _full(b1p.shape), _full(W2p.shape),
            _full(b2p.shape), _full(W3p.shape), _full(b3p.shape),
        ],
        out_specs=pl.BlockSpec((BE2, 2 * NT), lambda i: (i, 0)),
        out_shape=jax.ShapeDtypeStruct((E // 2, 2 * NT), jnp.float32),
    )(s_pair, eaTd, Wcp, b1p, W2p, b2p, W3p, b3p)


def _zero16(buf, nvec):
    def zb(i, _):
        buf[pl.ds(i * 16, 16)] = jnp.zeros((16,), jnp.float32)
        return _
    lax.fori_loop(0, nvec, zb, None)


def _sc_cnt(dst2d, ones2d):
    rpw = RE // 32
    CH = 4
    nchunk = rpw // CH
    slab = NPAD // 16

    @functools.partial(
        pl.kernel,
        out_type=jax.ShapeDtypeStruct((2 * NPAD,), jnp.float32),
        mesh=_sc_mesh(),
        compiler_params=pltpu.CompilerParams(use_tc_tiling_on_sc=False),
        scratch_types=[
            pltpu.VMEM_SHARED((NPAD,), jnp.float32),
            pltpu.VMEM((CH, 128), jnp.int32),
            pltpu.VMEM((CH, 128), jnp.float32),
            pltpu.VMEM((slab,), jnp.float32),
        ],
    )
    def k(dst_hbm, ones_hbm, out_hbm, acc, idx_v, val_v, zbuf):
        c = lax.axis_index("c")
        s = lax.axis_index("s")
        _zero16(zbuf, slab // 16)
        pltpu.sync_copy(zbuf, acc.at[pl.ds(s * slab, slab)])
        plsc.subcore_barrier()
        row0 = (c * 16 + s) * rpw

        def chunk(kk, _):
            r0 = row0 + kk * CH
            pltpu.sync_copy(dst_hbm.at[pl.ds(r0, CH)], idx_v)
            pltpu.sync_copy(ones_hbm.at[pl.ds(r0, CH)], val_v)
            for j in range(CH):
                pltpu.sync_copy(val_v.at[j], acc.at[idx_v.at[j]], add=True)
            return _

        lax.fori_loop(0, nchunk, chunk, None)
        plsc.subcore_barrier()
        pltpu.sync_copy(acc.at[pl.ds(s * slab, slab)], zbuf)
        pltpu.sync_copy(zbuf, out_hbm.at[pl.ds(c * NPAD + s * slab, slab)])

    return k(dst2d, ones2d)


def _sc_spmm(pk2, alpha_p, g_flat, n):
    rpw = RE // 16
    CH = 2
    nchunk = rpw // CH
    slab = NPAD // 16
    ZR = 224

    @functools.partial(
        pl.kernel,
        out_type=jax.ShapeDtypeStruct((2 * NPAD, 32), jnp.float32),
        mesh=_sc_mesh(),
        compiler_params=pltpu.CompilerParams(use_tc_tiling_on_sc=False),
        scratch_types=[
            pltpu.VMEM_SHARED((NPAD, 32), jnp.float32),
            pltpu.VMEM((2, CH, 2, 128), jnp.int32),
            pltpu.VMEM((2, CH, 128), jnp.float32),
            pltpu.VMEM((2, CH * 128, 32), jnp.float32),
            pltpu.SemaphoreType.DMA,
            pltpu.SemaphoreType.DMA,
            pltpu.SemaphoreType.DMA,
            pltpu.SemaphoreType.DMA,
        ],
    )
    def k(pk_hbm, alpha_hbm, g_hbm, out_hbm, acc, pk_v, alpha_v, msg,
          semI, semA, semG, semS):
        c = lax.axis_index("c")
        s = lax.axis_index("s")

        def zm(i, _):
            msg[0, i, pl.ds(0, 16)] = jnp.zeros((16,), jnp.float32)
            msg[0, i, pl.ds(16, 16)] = jnp.zeros((16,), jnp.float32)
            return _
        lax.fori_loop(0, ZR, zm, None)

        def zcp(t, _):
            pltpu.sync_copy(msg.at[0, pl.ds(0, ZR)],
                            acc.at[pl.ds(s * slab + t * ZR, ZR)])
            return _
        lax.fori_loop(0, 14, zcp, None)
        plsc.subcore_barrier()

        coff = jnp.full((16,), c * n, jnp.int32)
        row0 = s * rpw

        def issue_idx(kk, b):
            pltpu.async_copy(pk_hbm.at[pl.ds(row0 + kk * CH, CH)],
                             pk_v.at[b], semI)
            pltpu.async_copy(alpha_hbm.at[pl.ds(row0 + kk * CH, CH)],
                             alpha_v.at[b], semA)

        def wait_idx(b):
            pltpu.make_async_copy(pk_hbm.at[pl.ds(0, CH)], pk_v.at[b],
                                  semI).wait()
            pltpu.make_async_copy(alpha_hbm.at[pl.ds(0, CH)], alpha_v.at[b],
                                  semA).wait()

        def offs(b):
            def ofr(i, _):
                sl = pl.ds(i * 16, 16)
                for r in range(CH):
                    pk_v[b, r, 0, sl] = pk_v[b, r, 0, sl] + coff
                return _
            lax.fori_loop(0, 8, ofr, None)

        def issue_gather(b):
            for r in range(CH):
                pltpu.async_copy(g_hbm.at[pk_v.at[b, r, 0]],
                                 msg.at[b, pl.ds(r * 128, 128)], semG)

        def wait_gather(b):
            for r in range(CH):
                pltpu.make_async_copy(g_hbm.at[pk_v.at[b, r, 0]],
                                      msg.at[b, pl.ds(r * 128, 128)],
                                      semG).wait()

        def scale(b):
            for r in range(CH):
                def srow(gi, _):
                    a16 = alpha_v[b, r, pl.ds(gi * 16, 16)]
                    for j in range(16):
                        a = a16[jnp.full((16,), j, jnp.int32)]
                        e = r * 128 + gi * 16 + j
                        msg[b, e, pl.ds(0, 16)] = msg[b, e, pl.ds(0, 16)] * a
                        msg[b, e, pl.ds(16, 16)] = msg[b, e, pl.ds(16, 16)] * a
                    return _
                lax.fori_loop(0, 8, srow, None)

        def scatter_sync(b):
            descs = [pltpu.async_copy(msg.at[b, pl.ds(r * 128, 128)],
                                      acc.at[pk_v.at[b, r, 1]], semS,
                                      add=True)
                     for r in range(CH)]
            for dd in descs:
                dd.wait()

        issue_idx(0, 0)
        issue_idx(1, 1)
        wait_idx(0)
        offs(0)
        issue_gather(0)

        @pl.loop(0, nchunk, step=2)
        def chunk_loop(k0):
            for b in range(2):
                kk = k0 + b
                b1 = 1 - b

                @pl.when(kk + 1 < nchunk)
                def _():
                    wait_idx(b1)
                    offs(b1)
                    issue_gather(b1)

                wait_gather(b)
                scale(b)
                scatter_sync(b)

                @pl.when(kk + 2 < nchunk)
                def _():
                    issue_idx(kk + 2, b)

        plsc.subcore_barrier()

        def ecp(t, _):
            pltpu.sync_copy(acc.at[pl.ds(s * slab + t * ZR, ZR)],
                            msg.at[0, pl.ds(0, ZR)])
            pltpu.sync_copy(msg.at[0, pl.ds(0, ZR)],
                            out_hbm.at[pl.ds(c * NPAD + s * slab + t * ZR, ZR)])
            return _
        lax.fori_loop(0, 14, ecp, None)

    return k(pk2, alpha_p, g_flat)


def _sc_dec(pk2, pq_flat):
    rpw = RE // 32
    CH = 2
    nchunk = rpw // CH

    @functools.partial(
        pl.kernel,
        out_type=jax.ShapeDtypeStruct((RE * 128, HID), jnp.float32),
        mesh=_sc_mesh(),
        compiler_params=pltpu.CompilerParams(use_tc_tiling_on_sc=False),
        scratch_types=[
            pltpu.VMEM((2, CH, 2, 128), jnp.int32),
            pltpu.VMEM((2, CH * 128, HID), jnp.float32),
            pltpu.VMEM((2, CH * 128, HID), jnp.float32),
            pltpu.SemaphoreType.DMA,
            pltpu.SemaphoreType.DMA,
        ],
    )
    def k(pk_hbm, pq_hbm, out_hbm, pk_v, bufp, bufq, semI, semG):
        c = lax.axis_index("c")
        s = lax.axis_index("s")
        row0 = (c * 16 + s) * rpw

        def issue_idx(kk, b):
            pltpu.async_copy(pk_hbm.at[pl.ds(row0 + kk * CH, CH)],
                             pk_v.at[b], semI)

        def wait_idx(b):
            pltpu.make_async_copy(pk_hbm.at[pl.ds(0, CH)], pk_v.at[b],
                                  semI).wait()

        def issue_gather(b):
            for r in range(CH):
                pltpu.async_copy(pq_hbm.at[pk_v.at[b, r, 0]],
                                 bufp.at[b, pl.ds(r * 128, 128)], semG)
                pltpu.async_copy(pq_hbm.at[pk_v.at[b, r, 1]],
                                 bufq.at[b, pl.ds(r * 128, 128)], semG)

        def wait_gather(b):
            for r in range(CH):
                pltpu.make_async_copy(pq_hbm.at[pk_v.at[b, r, 0]],
                                      bufp.at[b, pl.ds(r * 128, 128)],
                                      semG).wait()
                pltpu.make_async_copy(pq_hbm.at[pk_v.at[b, r, 1]],
                                      bufq.at[b, pl.ds(r * 128, 128)],
                                      semG).wait()

        def addbuf(b):
            def addb(i, _):
                for t in range(4):
                    sl = pl.ds(t * 16, 16)
                    bufp[b, i, sl] = bufp[b, i, sl] + bufq[b, i, sl]
                return _
            lax.fori_loop(0, CH * 128, addb, None)

        issue_idx(0, 0)
        issue_idx(1, 1)
        wait_idx(0)
        issue_gather(0)

        @pl.loop(0, nchunk, step=2)
        def chunk_loop(k0):
            for b in range(2):
                kk = k0 + b
                b1 = 1 - b

                @pl.when(kk + 1 < nchunk)
                def _():
                    wait_idx(b1)
                    issue_gather(b1)

                wait_gather(b)
                addbuf(b)
                pltpu.sync_copy(
                    bufp.at[b],
                    out_hbm.at[pl.ds((row0 + kk * CH) * 128, CH * 128)])

                @pl.when(kk + 2 < nchunk)
                def _():
                    issue_idx(kk + 2, b)

    return k(pk2, pq_flat)


def kernel(x, edge_index, edge_attr, u,
           W_in, b_in, W_e1, b_e1, W_e2, b_e2, W_m, b_m,
           W_ih, b_ih, W_hh, b_hh, W_d1, b_d1, W_d2, b_d2, W_d3, b_d3):
    n = x.shape[0]
    E = edge_attr.shape[0]
    src, dst = edge_index[0], edge_index[1]

    A2 = W_e1[:, :EDGE_DIM]
    c1c = (b_e1 + (u @ W_e1[:, EDGE_DIM:].T)[0]).reshape(-1, 1)
    w2c = W_e2.T
    b2 = b_e2.reshape(1, 1)
    XT = W_in[:, :NODE_DIM].T
    c_in = (b_in + (u @ W_in[:, NODE_DIM:].T)[0]).reshape(1, HID)
    WmT = W_m.T
    bm = b_m.reshape(1, HID)
    WihT0 = W_ih[:, :32].T
    WihT1 = W_ih[:, 32:].T
    bih = b_ih.reshape(1, 3 * HID)
    WhhT = W_hh.T
    bhh = b_hh.reshape(1, 3 * HID)
    Wd1aT = W_d1[:, :HID].T
    Wd1bT = W_d1[:, HID:2 * HID].T
    WcT = W_d1[:, 2 * HID:].T
    bd1 = b_d1.reshape(1, HID)
    Wd2T = W_d2.T
    bd2 = b_d2.reshape(1, -1)
    Wd3T = W_d3.T
    bd3 = b_d3.reshape(1, -1)
    zc = jnp.zeros_like(WcT)
    Wcp = jnp.block([[WcT, zc], [zc, WcT]])
    b1p = jnp.concatenate([bd1, bd1], axis=1)
    z2 = jnp.zeros_like(Wd2T)
    W2p = jnp.block([[Wd2T, z2], [z2, Wd2T]])
    b2p = jnp.concatenate([bd2, bd2], axis=1)
    z3 = jnp.zeros_like(Wd3T)
    W3p = jnp.block([[Wd3T, z3], [z3, Wd3T]])
    b3p = jnp.concatenate([bd3, bd3], axis=1)

    pad = RE * 128 - E
    src_p = jnp.pad(src, (0, pad)).reshape(RE, 128)
    dst_p = jnp.pad(dst, (0, pad)).reshape(RE, 128)
    dstn_p = jnp.pad(dst + n, (0, pad)).reshape(RE, 128)
    ones_p = jnp.pad(jnp.ones((E,), jnp.float32), (0, pad)).reshape(RE, 128)

    eaT = edge_attr.T
    alpha = _alpha_kernel(eaT, A2, c1c, w2c, b2)
    alpha_p = jnp.pad(alpha[0], (0, pad)).reshape(RE, 128)
    pk_sd = jnp.stack([src_p, dst_p], axis=1)
    pk2 = jnp.stack([src_p, dstn_p], axis=1)

    cnt2 = _sc_cnt(dst_p, ones_p).reshape(2, NPAD)
    cnt0 = cnt2[0, :n].reshape(n, 1)
    cnt1 = cnt2[1, :n].reshape(n, 1)
    h, g, recip = _node0_kernel(x, cnt0, cnt1, XT, c_in, WmT, bm)

    for step in range(STEPS):
        g_flat = g.reshape(2 * n, 32)
        aggp = _sc_spmm(pk_sd, alpha_p, g_flat, n).reshape(2, NPAD, 32)
        final = step == STEPS - 1
        outs = _gru_kernel(h, aggp, recip, WihT0, WihT1, bih, WhhT, bhh,
                           WmT, bm, Wd1aT, Wd1bT, final)
        if final:
            pq, = outs
        else:
            h, g = outs

    pq_flat = pq.reshape(2 * n, HID)
    s = _sc_dec(pk2, pq_flat)
    s_pair = s.reshape(RE * 64, 128)
    eaTd = jnp.transpose(eaT.reshape(EDGE_DIM, E // 2, 2),
                         (2, 0, 1)).reshape(2 * EDGE_DIM, E // 2)
    outp = _dec2_kernel(s_pair, eaTd, Wcp, b1p, W2p, b2p, W3p, b3p, E)
    return outp.reshape(E, NT)


# speedup vs baseline: 6.5901x; 1.0913x over previous
import functools

import jax
import jax.numpy as jnp
from jax import lax
from jax.experimental import pallas as pl
from jax.experimental.pallas import tpu as pltpu
from jax.experimental.pallas import tpu_sc as plsc

NODE_DIM, EDGE_DIM, GLOBAL_DIM, HID, NT, STEPS = 12, 5, 11, 64, 4, 3
BN = 2000
BE = 3200
RE = 6272
NPAD = 50176


def _full(shape):
    return pl.BlockSpec(shape, lambda i: tuple(0 for _ in shape))


def _sc_mesh():
    return plsc.VectorSubcoreMesh(core_axis_name="c", subcore_axis_name="s")


def _alpha_kernel(eaT, A2, c1c, w2c, b2):
    E = eaT.shape[1]
    ABE = 16000

    def body(eaT_ref, A2_ref, c1_ref, w2_ref, b2_ref, out_ref):
        t = jnp.dot(A2_ref[...], eaT_ref[...], preferred_element_type=jnp.float32)
        t = jnp.maximum(t + c1_ref[...], 0.0)
        z = jnp.sum(t * w2_ref[...], axis=0, keepdims=True) + b2_ref[...]
        out_ref[...] = jax.nn.sigmoid(z)

    return pl.pallas_call(
        body,
        grid=(E // ABE,),
        in_specs=[
            pl.BlockSpec((EDGE_DIM, ABE), lambda i: (0, i)),
            _full(A2.shape), _full(c1c.shape), _full(w2c.shape), _full(b2.shape),
        ],
        out_specs=pl.BlockSpec((1, ABE), lambda i: (0, i)),
        out_shape=jax.ShapeDtypeStruct((1, E), jnp.float32),
    )(eaT, A2, c1c, w2c, b2)


def _node0_kernel(x, cnt0, cnt1, XT, c_in, WmT, b_m):
    n = x.shape[0]

    def body(x_ref, c0_ref, c1r_ref, XT_ref, cin_ref, WmT_ref, bm_ref,
             h_ref, g_ref, recip_ref):
        h = jnp.tanh(jnp.dot(x_ref[...], XT_ref[...],
                             preferred_element_type=jnp.float32) + cin_ref[...])
        g = jnp.maximum(jnp.dot(h, WmT_ref[...],
                                preferred_element_type=jnp.float32) + bm_ref[...], 0.0)
        h_ref[...] = h
        g_ref[0] = g[:, :32]
        g_ref[1] = g[:, 32:]
        cnt = c0_ref[...] + c1r_ref[...]
        recip_ref[...] = 1.0 / jnp.maximum(cnt, 1.0)

    return pl.pallas_call(
        body,
        grid=(n // BN,),
        in_specs=[
            pl.BlockSpec((BN, NODE_DIM), lambda i: (i, 0)),
            pl.BlockSpec((BN, 1), lambda i: (i, 0)),
            pl.BlockSpec((BN, 1), lambda i: (i, 0)),
            _full(XT.shape), _full(c_in.shape), _full(WmT.shape), _full(b_m.shape),
        ],
        out_specs=[
            pl.BlockSpec((BN, HID), lambda i: (i, 0)),
            pl.BlockSpec((2, BN, 32), lambda i: (0, i, 0)),
            pl.BlockSpec((BN, 1), lambda i: (i, 0)),
        ],
        out_shape=[
            jax.ShapeDtypeStruct((n, HID), jnp.float32),
            jax.ShapeDtypeStruct((2, n, 32), jnp.float32),
            jax.ShapeDtypeStruct((n, 1), jnp.float32),
        ],
    )(x, cnt0, cnt1, XT, c_in, WmT, b_m)


def _gru_kernel(h, aggp, recip, WihT0, WihT1, b_ih, WhhT, b_hh, WmT, b_m,
                Wd1aT, Wd1bT, final):
    n = h.shape[0]

    def body(h_ref, agg_ref, recip_ref, WihT0_ref, WihT1_ref, bih_ref,
             WhhT_ref, bhh_ref, WmT_ref, bm_ref, Wa_ref, Wb_ref, *outs):
        r_ = recip_ref[...]
        a0 = agg_ref[0] * r_
        a1 = agg_ref[1] * r_
        gi = (jnp.dot(a0, WihT0_ref[...], preferred_element_type=jnp.float32)
              + jnp.dot(a1, WihT1_ref[...], preferred_element_type=jnp.float32)
              + bih_ref[...])
        h = h_ref[...]
        gh = jnp.dot(h, WhhT_ref[...], preferred_element_type=jnp.float32) + bhh_ref[...]
        r = jax.nn.sigmoid(gi[:, :HID] + gh[:, :HID])
        z = jax.nn.sigmoid(gi[:, HID:2 * HID] + gh[:, HID:2 * HID])
        nn_ = jnp.tanh(gi[:, 2 * HID:] + r * gh[:, 2 * HID:])
        h_new = (1.0 - z) * nn_ + z * h
        if final:
            pq_ref, = outs
            pq_ref[0] = jnp.dot(h_new, Wa_ref[...], preferred_element_type=jnp.float32)
            pq_ref[1] = jnp.dot(h_new, Wb_ref[...], preferred_element_type=jnp.float32)
        else:
            hn_ref, g_ref = outs
            hn_ref[...] = h_new
            g = jnp.maximum(jnp.dot(h_new, WmT_ref[...],
                                    preferred_element_type=jnp.float32) + bm_ref[...], 0.0)
            g_ref[0] = g[:, :32]
            g_ref[1] = g[:, 32:]

    if final:
        out_specs = [pl.BlockSpec((2, BN, HID), lambda i: (0, i, 0))]
        out_shape = [jax.ShapeDtypeStruct((2, n, HID), jnp.float32)]
    else:
        out_specs = [
            pl.BlockSpec((BN, HID), lambda i: (i, 0)),
            pl.BlockSpec((2, BN, 32), lambda i: (0, i, 0)),
        ]
        out_shape = [
            jax.ShapeDtypeStruct((n, HID), jnp.float32),
            jax.ShapeDtypeStruct((2, n, 32), jnp.float32),
        ]

    return pl.pallas_call(
        body,
        grid=(n // BN,),
        in_specs=[
            pl.BlockSpec((BN, HID), lambda i: (i, 0)),
            pl.BlockSpec((2, BN, 32), lambda i: (0, i, 0)),
            pl.BlockSpec((BN, 1), lambda i: (i, 0)),
            _full(WihT0.shape), _full(WihT1.shape), _full(b_ih.shape),
            _full(WhhT.shape), _full(b_hh.shape), _full(WmT.shape),
            _full(b_m.shape), _full(Wd1aT.shape), _full(Wd1bT.shape),
        ],
        out_specs=out_specs,
        out_shape=out_shape,
    )(h, aggp, recip, WihT0, WihT1, b_ih, WhhT, b_hh, WmT, b_m, Wd1aT, Wd1bT)


def _dec2_kernel(s_pair, eaTd, Wcp, b1p, W2p, b2p, W3p, b3p, E):
    BE2 = 3200

    def body(s_ref, ead_ref, Wcp_ref, b1_ref, W2_ref, b2_ref, W3_ref, b3_ref,
             out_ref):
        rp = lax.dot_general(ead_ref[...], Wcp_ref[...],
                             (((0,), (0,)), ((), ())),
                             preferred_element_type=jnp.float32)
        d1 = jnp.maximum(s_ref[...] + rp + b1_ref[...], 0.0)
        d2 = jnp.maximum(jnp.dot(d1, W2_ref[...],
                                 preferred_element_type=jnp.float32) + b2_ref[...], 0.0)
        out_ref[...] = jnp.dot(d2, W3_ref[...],
                               preferred_element_type=jnp.float32) + b3_ref[...]

    return pl.pallas_call(
        body,
        grid=(E // (2 * BE2),),
        in_specs=[
            pl.BlockSpec((BE2, 128), lambda i: (i, 0)),
            pl.BlockSpec((2 * EDGE_DIM, BE2), lambda i: (0, i)),
            _full(Wcp.shape), _full(b1p.shape), _full(W2p.shape),
            _full(b2p.shape), _full(W3p.shape), _full(b3p.shape),
        ],
        out_specs=pl.BlockSpec((BE2, 2 * NT), lambda i: (i, 0)),
        out_shape=jax.ShapeDtypeStruct((E // 2, 2 * NT), jnp.float32),
    )(s_pair, eaTd, Wcp, b1p, W2p, b2p, W3p, b3p)


def _zero16(buf, nvec):
    def zb(i, _):
        buf[pl.ds(i * 16, 16)] = jnp.zeros((16,), jnp.float32)
        return _
    lax.fori_loop(0, nvec, zb, None)


def _sc_cnt(pk, ones2d):
    rpw = RE // 32
    CH = 4
    nchunk = rpw // CH
    slab = NPAD // 16

    @functools.partial(
        pl.kernel,
        out_type=jax.ShapeDtypeStruct((2 * NPAD,), jnp.float32),
        mesh=_sc_mesh(),
        compiler_params=pltpu.CompilerParams(use_tc_tiling_on_sc=False),
        scratch_types=[
            pltpu.VMEM_SHARED((NPAD,), jnp.float32),
            pltpu.VMEM((CH, 2, 128), jnp.int32),
            pltpu.VMEM((CH, 128), jnp.float32),
            pltpu.VMEM((slab,), jnp.float32),
        ],
    )
    def k(pk_hbm, ones_hbm, out_hbm, acc, idx_v, val_v, zbuf):
        c = lax.axis_index("c")
        s = lax.axis_index("s")
        _zero16(zbuf, slab // 16)
        pltpu.sync_copy(zbuf, acc.at[pl.ds(s * slab, slab)])
        plsc.subcore_barrier()
        row0 = (c * 16 + s) * rpw

        def chunk(kk, _):
            r0 = row0 + kk * CH
            pltpu.sync_copy(pk_hbm.at[pl.ds(r0, CH)], idx_v)
            pltpu.sync_copy(ones_hbm.at[pl.ds(r0, CH)], val_v)
            for j in range(CH):
                pltpu.sync_copy(val_v.at[j], acc.at[idx_v.at[j, 1]], add=True)
            return _

        lax.fori_loop(0, nchunk, chunk, None)
        plsc.subcore_barrier()
        pltpu.sync_copy(acc.at[pl.ds(s * slab, slab)], zbuf)
        pltpu.sync_copy(zbuf, out_hbm.at[pl.ds(c * NPAD + s * slab, slab)])

    return k(pk, ones2d)


def _sc_spmm(pk2, alpha_p, g_flat, n):
    rpw = RE // 16
    CH = 2
    nchunk = rpw // CH
    slab = NPAD // 16
    ZR = 224

    @functools.partial(
        pl.kernel,
        out_type=jax.ShapeDtypeStruct((2 * NPAD, 32), jnp.float32),
        mesh=_sc_mesh(),
        compiler_params=pltpu.CompilerParams(use_tc_tiling_on_sc=False),
        scratch_types=[
            pltpu.VMEM_SHARED((NPAD, 32), jnp.float32),
            pltpu.VMEM((2, CH, 2, 128), jnp.int32),
            pltpu.VMEM((2, CH, 128), jnp.float32),
            pltpu.VMEM((2, CH * 128, 32), jnp.float32),
            pltpu.SemaphoreType.DMA,
            pltpu.SemaphoreType.DMA,
            pltpu.SemaphoreType.DMA,
            pltpu.SemaphoreType.DMA,
        ],
    )
    def k(pk_hbm, alpha_hbm, g_hbm, out_hbm, acc, pk_v, alpha_v, msg,
          semI, semA, semG, semS):
        c = lax.axis_index("c")
        s = lax.axis_index("s")

        def zm(i, _):
            msg[0, i, pl.ds(0, 16)] = jnp.zeros((16,), jnp.float32)
            msg[0, i, pl.ds(16, 16)] = jnp.zeros((16,), jnp.float32)
            return _
        lax.fori_loop(0, ZR, zm, None)

        def zcp(t, _):
            pltpu.sync_copy(msg.at[0, pl.ds(0, ZR)],
                            acc.at[pl.ds(s * slab + t * ZR, ZR)])
            return _
        lax.fori_loop(0, 14, zcp, None)
        plsc.subcore_barrier()

        coff = jnp.full((16,), c * n, jnp.int32)
        row0 = s * rpw

        def issue_idx(kk, b):
            pltpu.async_copy(pk_hbm.at[pl.ds(row0 + kk * CH, CH)],
                             pk_v.at[b], semI)
            pltpu.async_copy(alpha_hbm.at[pl.ds(row0 + kk * CH, CH)],
                             alpha_v.at[b], semA)

        def wait_idx(b):
            pltpu.make_async_copy(pk_hbm.at[pl.ds(0, CH)], pk_v.at[b],
                                  semI).wait()
            pltpu.make_async_copy(alpha_hbm.at[pl.ds(0, CH)], alpha_v.at[b],
                                  semA).wait()

        def offs(b):
            def ofr(i, _):
                sl = pl.ds(i * 16, 16)
                for r in range(CH):
                    pk_v[b, r, 0, sl] = pk_v[b, r, 0, sl] + coff
                return _
            lax.fori_loop(0, 8, ofr, None)

        def issue_gather(b):
            for r in range(CH):
                pltpu.async_copy(g_hbm.at[pk_v.at[b, r, 0]],
                                 msg.at[b, pl.ds(r * 128, 128)], semG)

        def wait_gather(b):
            for r in range(CH):
                pltpu.make_async_copy(g_hbm.at[pk_v.at[b, r, 0]],
                                      msg.at[b, pl.ds(r * 128, 128)],
                                      semG).wait()

        def scale(b):
            for r in range(CH):
                def srow(gi, _):
                    a16 = alpha_v[b, r, pl.ds(gi * 16, 16)]
                    for j in range(16):
                        a = a16[jnp.full((16,), j, jnp.int32)]
                        e = r * 128 + gi * 16 + j
                        msg[b, e, pl.ds(0, 16)] = msg[b, e, pl.ds(0, 16)] * a
                        msg[b, e, pl.ds(16, 16)] = msg[b, e, pl.ds(16, 16)] * a
                    return _
                lax.fori_loop(0, 8, srow, None)

        def scatter_sync(b):
            descs = [pltpu.async_copy(msg.at[b, pl.ds(r * 128, 128)],
                                      acc.at[pk_v.at[b, r, 1]], semS,
                                      add=True)
                     for r in range(CH)]
            for dd in descs:
                dd.wait()

        issue_idx(0, 0)
        issue_idx(1, 1)
        wait_idx(0)
        offs(0)
        issue_gather(0)

        @pl.loop(0, nchunk, step=2)
        def chunk_loop(k0):
            for b in range(2):
                kk = k0 + b
                b1 = 1 - b

                @pl.when(kk + 1 < nchunk)
                def _():
                    wait_idx(b1)
                    offs(b1)
                    issue_gather(b1)

                wait_gather(b)
                scale(b)
                scatter_sync(b)

                @pl.when(kk + 2 < nchunk)
                def _():
                    issue_idx(kk + 2, b)

        plsc.subcore_barrier()

        def ecp(t, _):
            pltpu.sync_copy(acc.at[pl.ds(s * slab + t * ZR, ZR)],
                            msg.at[0, pl.ds(0, ZR)])
            pltpu.sync_copy(msg.at[0, pl.ds(0, ZR)],
                            out_hbm.at[pl.ds(c * NPAD + s * slab + t * ZR, ZR)])
            return _
        lax.fori_loop(0, 14, ecp, None)

    return k(pk2, alpha_p, g_flat)


def _sc_dec(pk, pq_flat, n):
    rpw = RE // 32
    CH = 2
    nchunk = rpw // CH

    @functools.partial(
        pl.kernel,
        out_type=jax.ShapeDtypeStruct((RE * 128, HID), jnp.float32),
        mesh=_sc_mesh(),
        compiler_params=pltpu.CompilerParams(use_tc_tiling_on_sc=False),
        scratch_types=[
            pltpu.VMEM((2, CH, 2, 128), jnp.int32),
            pltpu.VMEM((2, CH * 128, HID), jnp.float32),
            pltpu.VMEM((2, CH * 128, HID), jnp.float32),
            pltpu.SemaphoreType.DMA,
            pltpu.SemaphoreType.DMA,
        ],
    )
    def k(pk_hbm, pq_hbm, out_hbm, pk_v, bufp, bufq, semI, semG):
        c = lax.axis_index("c")
        s = lax.axis_index("s")
        row0 = (c * 16 + s) * rpw
        nvec = jnp.full((16,), n, jnp.int32)

        def issue_idx(kk, b):
            pltpu.async_copy(pk_hbm.at[pl.ds(row0 + kk * CH, CH)],
                             pk_v.at[b], semI)

        def wait_idx(b):
            pltpu.make_async_copy(pk_hbm.at[pl.ds(0, CH)], pk_v.at[b],
                                  semI).wait()

        def offs(b):
            def ofr(i, _):
                sl = pl.ds(i * 16, 16)
                for r in range(CH):
                    pk_v[b, r, 1, sl] = pk_v[b, r, 1, sl] + nvec
                return _
            lax.fori_loop(0, 8, ofr, None)

        def issue_gather(b):
            for r in range(CH):
                pltpu.async_copy(pq_hbm.at[pk_v.at[b, r, 0]],
                                 bufp.at[b, pl.ds(r * 128, 128)], semG)
                pltpu.async_copy(pq_hbm.at[pk_v.at[b, r, 1]],
                                 bufq.at[b, pl.ds(r * 128, 128)], semG)

        def wait_gather(b):
            for r in range(CH):
                pltpu.make_async_copy(pq_hbm.at[pk_v.at[b, r, 0]],
                                      bufp.at[b, pl.ds(r * 128, 128)],
                                      semG).wait()
                pltpu.make_async_copy(pq_hbm.at[pk_v.at[b, r, 1]],
                                      bufq.at[b, pl.ds(r * 128, 128)],
                                      semG).wait()

        def addbuf(b):
            def addb(i, _):
                for t in range(4):
                    sl = pl.ds(t * 16, 16)
                    bufp[b, i, sl] = bufp[b, i, sl] + bufq[b, i, sl]
                return _
            lax.fori_loop(0, CH * 128, addb, None)

        issue_idx(0, 0)
        issue_idx(1, 1)
        wait_idx(0)
        offs(0)
        issue_gather(0)

        @pl.loop(0, nchunk, step=2)
        def chunk_loop(k0):
            for b in range(2):
                kk = k0 + b
                b1 = 1 - b

                @pl.when(kk + 1 < nchunk)
                def _():
                    wait_idx(b1)
                    offs(b1)
                    issue_gather(b1)

                wait_gather(b)
                addbuf(b)
                pltpu.sync_copy(
                    bufp.at[b],
                    out_hbm.at[pl.ds((row0 + kk * CH) * 128, CH * 128)])

                @pl.when(kk + 2 < nchunk)
                def _():
                    issue_idx(kk + 2, b)

    return k(pk, pq_flat)


def kernel(x, edge_index, edge_attr, u,
           W_in, b_in, W_e1, b_e1, W_e2, b_e2, W_m, b_m,
           W_ih, b_ih, W_hh, b_hh, W_d1, b_d1, W_d2, b_d2, W_d3, b_d3):
    n = x.shape[0]
    E = edge_attr.shape[0]

    A2 = W_e1[:, :EDGE_DIM]
    c1c = (b_e1 + (u @ W_e1[:, EDGE_DIM:].T)[0]).reshape(-1, 1)
    w2c = W_e2.T
    b2 = b_e2.reshape(1, 1)
    XT = W_in[:, :NODE_DIM].T
    c_in = (b_in + (u @ W_in[:, NODE_DIM:].T)[0]).reshape(1, HID)
    WmT = W_m.T
    bm = b_m.reshape(1, HID)
    WihT0 = W_ih[:, :32].T
    WihT1 = W_ih[:, 32:].T
    bih = b_ih.reshape(1, 3 * HID)
    WhhT = W_hh.T
    bhh = b_hh.reshape(1, 3 * HID)
    Wd1aT = W_d1[:, :HID].T
    Wd1bT = W_d1[:, HID:2 * HID].T
    WcT = W_d1[:, 2 * HID:].T
    bd1 = b_d1.reshape(1, HID)
    Wd2T = W_d2.T
    bd2 = b_d2.reshape(1, -1)
    Wd3T = W_d3.T
    bd3 = b_d3.reshape(1, -1)
    zc = jnp.zeros_like(WcT)
    Wcp = jnp.block([[WcT, zc], [zc, WcT]])
    b1p = jnp.concatenate([bd1, bd1], axis=1)
    z2 = jnp.zeros_like(Wd2T)
    W2p = jnp.block([[Wd2T, z2], [z2, Wd2T]])
    b2p = jnp.concatenate([bd2, bd2], axis=1)
    z3 = jnp.zeros_like(Wd3T)
    W3p = jnp.block([[Wd3T, z3], [z3, Wd3T]])
    b3p = jnp.concatenate([bd3, bd3], axis=1)

    pad = RE * 128 - E
    pk = jnp.pad(jnp.transpose(edge_index.reshape(2, E // 128, 128),
                               (1, 0, 2)),
                 ((0, RE - E // 128), (0, 0), (0, 0)))
    ones_p = jnp.pad(jnp.ones((E,), jnp.float32), (0, pad)).reshape(RE, 128)

    eaT = edge_attr.T
    alpha = _alpha_kernel(eaT, A2, c1c, w2c, b2)
    alpha_p = jnp.pad(alpha[0], (0, pad)).reshape(RE, 128)

    cnt2 = _sc_cnt(pk, ones_p).reshape(2, NPAD)
    cnt0 = cnt2[0, :n].reshape(n, 1)
    cnt1 = cnt2[1, :n].reshape(n, 1)
    h, g, recip = _node0_kernel(x, cnt0, cnt1, XT, c_in, WmT, bm)

    for step in range(STEPS):
        g_flat = g.reshape(2 * n, 32)
        aggp = _sc_spmm(pk, alpha_p, g_flat, n).reshape(2, NPAD, 32)
        final = step == STEPS - 1
        outs = _gru_kernel(h, aggp, recip, WihT0, WihT1, bih, WhhT, bhh,
                           WmT, bm, Wd1aT, Wd1bT, final)
        if final:
            pq, = outs
        else:
            h, g = outs

    pq_flat = pq.reshape(2 * n, HID)
    s = _sc_dec(pk, pq_flat, n)
    s_pair = s.reshape(RE * 64, 128)
    eaTd = jnp.transpose(eaT.reshape(EDGE_DIM, E // 2, 2),
                         (2, 0, 1)).reshape(2 * EDGE_DIM, E // 2)
    outp = _dec2_kernel(s_pair, eaTd, Wcp, b1p, W2p, b2p, W3p, b3p, E)
    return outp.reshape(E, NT)
